# baseline (speedup 1.0000x reference)
import functools

import jax
import jax.numpy as jnp
from jax import lax
from jax.experimental import pallas as pl
from jax.experimental.pallas import tpu as pltpu
from jax.experimental.pallas import tpu_sc as plsc

_B, _NB, _T, _D, _K = 4, 96, 256, 64, 8192
_KC = 2048


def _mm(a, b):
    return lax.dot_general(a, b, (((1,), (0,)), ((), ())),
                           preferred_element_type=jnp.float32)


def _mm_t(a, b):
    return lax.dot_general(a, b, (((1,), (1,)), ((), ())),
                           preferred_element_type=jnp.float32)


def _conv1d(x, w, b):
    y = lax.conv_general_dilated(x, w, window_strides=(1,), padding='SAME',
                                 dimension_numbers=('NCH', 'OIH', 'NCH'))
    return y + b[None, :, None]


def _attn_blk(x, a):
    xt = jnp.transpose(x, (0, 2, 1))
    q = xt @ a['wq'] + a['bq']
    k = xt @ a['wk'] + a['bk']
    v = xt @ a['wv'] + a['bv']
    scale = jnp.sqrt(jnp.asarray(q.shape[-1], dtype=x.dtype))
    attn = jax.nn.softmax(q @ jnp.transpose(k, (0, 2, 1)) / scale, axis=-1)
    o = (attn @ v) @ a['wo'] + a['bo']
    return x + jnp.transpose(o, (0, 2, 1))


def _encode(x, p):
    z = x * p['w_proj'][None, :, None]
    for w, b in p['enc_conv']:
        z = jax.nn.relu(_conv1d(z, w, b))
    for a in p['enc_attn']:
        z = _attn_blk(z, a)
    return z


def _vq_body(zt_ref, emb_ref, e2_ref, idx_ref):
    f = zt_ref[0]
    f2 = jnp.sum(f * f, axis=1, keepdims=True)

    def chunk(j, carry):
        bd, bi = carry
        e = emb_ref[pl.ds(j * _KC, _KC), :]
        g = _mm_t(f, e)
        e2 = e2_ref[0, pl.ds(j * _KC, _KC)][None, :]
        d = (f2 - 2.0 * g) + e2
        dmin = jnp.min(d, axis=1, keepdims=True)
        ids = lax.broadcasted_iota(jnp.int32, (_T, _KC), 1) + j * _KC
        imin = jnp.min(jnp.where(d == dmin, ids, jnp.int32(2**31 - 1)),
                       axis=1, keepdims=True)
        better = dmin < bd
        return jnp.where(better, dmin, bd), jnp.where(better, imin, bi)

    bd0 = jnp.full((_T, 1), jnp.inf, jnp.float32)
    bi0 = jnp.zeros((_T, 1), jnp.int32)
    _, bi = lax.fori_loop(0, _K // _KC, chunk, (bd0, bi0))
    idx_ref[0] = bi


def _gather_sc(table, idx):
    dp = 128
    tablep = jnp.pad(table, ((0, 0), (0, dp - table.shape[1])))
    info = plsc.get_sparse_core_info()
    nw = info.num_cores * info.num_subcores
    bt = idx.shape[0]
    bpw = bt // nw
    nc = info.num_cores
    mesh = plsc.VectorSubcoreMesh(core_axis_name="c", subcore_axis_name="s")

    @functools.partial(
        pl.kernel, mesh=mesh,
        out_type=jax.ShapeDtypeStruct((bt, dp), jnp.float32),
        scratch_types=[
            pltpu.VMEM((bpw,), jnp.int32),
            pltpu.VMEM((bpw, dp), jnp.float32),
            pltpu.SemaphoreType.DMA,
        ],
    )
    def k(table_hbm, idx_hbm, out_hbm, idx_v, rows_v, sem):
        wid = lax.axis_index("s") * nc + lax.axis_index("c")
        base = wid * bpw
        pltpu.sync_copy(idx_hbm.at[pl.ds(base, bpw)], idx_v)
        pltpu.async_copy(table_hbm.at[idx_v], rows_v, sem).wait()
        pltpu.sync_copy(rows_v, out_hbm.at[pl.ds(base, bpw)])

    return k(tablep, idx)[:, :_D]


def _conv3(h, w0, w1, w2, b):
    zrow = jnp.zeros((1, h.shape[1]), h.dtype)
    prev = jnp.concatenate([zrow, h[:-1]], axis=0)
    nxt = jnp.concatenate([h[1:], zrow], axis=0)
    y = _mm(prev, w0) + _mm(h, w1) + _mm(nxt, w2) + b
    return jnp.maximum(y, 0.0)


def _attn(h, wq, bq, wk, bk, wv, bv, wo, bo):
    q = _mm(h, wq) + bq
    k = _mm(h, wk) + bk
    v = _mm(h, wv) + bv
    s = _mm_t(q, k) * (1.0 / 8.0)
    m = jnp.max(s, axis=1, keepdims=True)
    e = jnp.exp(s - m)
    a = e / jnp.sum(e, axis=1, keepdims=True)
    o = _mm(_mm(a, v), wo) + bo
    return h + o


def _dec_body(zqt_ref,
              a0wq, a0bq, a0wk, a0bk, a0wv, a0bv, a0wo, a0bo,
              a1wq, a1bq, a1wk, a1bk, a1wv, a1bv, a1wo, a1bo,
              d1w0, d1w1, d1w2, d1b,
              d2w0, d2w1, d2w2, d2b,
              d3w0, d3w1, d3w2, d3b,
              rw_ref, rb_ref,
              out_ref):
    h = zqt_ref[0]
    h = _attn(h, a0wq[...], a0bq[...], a0wk[...], a0bk[...],
              a0wv[...], a0bv[...], a0wo[...], a0bo[...])
    h = _attn(h, a1wq[...], a1bq[...], a1wk[...], a1bk[...],
              a1wv[...], a1bv[...], a1wo[...], a1bo[...])
    h = _conv3(h, d1w0[...], d1w1[...], d1w2[...], d1b[...])
    h = _conv3(h, d2w0[...], d2w1[...], d2w2[...], d2b[...])
    h = _conv3(h, d3w0[...], d3w1[...], d3w2[...], d3b[...])
    out_ref[0] = _mm(h, rw_ref[...]) + rb_ref[...]


def _full_spec(arr):
    nd = arr.ndim
    return pl.BlockSpec(arr.shape, lambda b, _nd=nd: (0,) * _nd)


def _attn_flat(a):
    return [a['wq'], a['bq'].reshape(1, -1), a['wk'], a['bk'].reshape(1, -1),
            a['wv'], a['bv'].reshape(1, -1), a['wo'], a['bo'].reshape(1, -1)]


def _conv_flat(wb):
    w, b = wb
    return [w[:, :, 0].T, w[:, :, 1].T, w[:, :, 2].T, b.reshape(1, -1)]


def kernel(x, params):
    p = params
    emb = p['emb']

    z = _encode(x, p)
    zt = jnp.transpose(z, (0, 2, 1))
    e2 = jnp.sum(emb**2, axis=1)[None, :]

    idx3 = pl.pallas_call(
        _vq_body,
        grid=(_B,),
        in_specs=[pl.BlockSpec((1, _T, _D), lambda b: (b, 0, 0)),
                  _full_spec(emb), _full_spec(e2)],
        out_specs=pl.BlockSpec((1, _T, 1), lambda b: (b, 0, 0)),
        out_shape=jax.ShapeDtypeStruct((_B, _T, 1), jnp.int32),
    )(zt, emb, e2)

    indices = idx3.reshape(_B, _T)
    zq_flat = _gather_sc(emb, idx3.reshape(-1))
    zqt = zq_flat.reshape(_B, _T, _D)

    dec_in = [zqt]
    for a in p['dec_attn']:
        dec_in += _attn_flat(a)
    for wb in p['dec_conv']:
        dec_in += _conv_flat(wb)
    dec_in += [p['rw'], p['rb'].reshape(1, -1)]

    recont = pl.pallas_call(
        _dec_body,
        grid=(_B,),
        in_specs=[pl.BlockSpec((1, _T, _D), lambda b: (b, 0, 0))]
                 + [_full_spec(a) for a in dec_in[1:]],
        out_specs=pl.BlockSpec((1, _T, _NB), lambda b: (b, 0, 0)),
        out_shape=jax.ShapeDtypeStruct((_B, _T, _NB), jnp.float32),
    )(*dec_in)

    recon = jnp.transpose(recont, (0, 2, 1))
    zq = jnp.transpose(zqt, (0, 2, 1))
    return recon, z, zq, indices


# speedup vs baseline: 1.1796x; 1.1796x over previous
import functools

import jax
import jax.numpy as jnp
from jax import lax
from jax.experimental import pallas as pl
from jax.experimental.pallas import tpu as pltpu
from jax.experimental.pallas import tpu_sc as plsc

_B, _NB, _T, _D, _K = 4, 96, 256, 64, 8192
_KC = 2048


def _mm(a, b):
    return lax.dot_general(a, b, (((1,), (0,)), ((), ())),
                           preferred_element_type=jnp.float32)


def _mm_t(a, b):
    return lax.dot_general(a, b, (((1,), (1,)), ((), ())),
                           preferred_element_type=jnp.float32)


def _conv1d(x, w, b):
    y = lax.conv_general_dilated(x, w, window_strides=(1,), padding='SAME',
                                 dimension_numbers=('NCH', 'OIH', 'NCH'))
    return y + b[None, :, None]


def _attn_blk(x, a):
    xt = jnp.transpose(x, (0, 2, 1))
    q = xt @ a['wq'] + a['bq']
    k = xt @ a['wk'] + a['bk']
    v = xt @ a['wv'] + a['bv']
    scale = jnp.sqrt(jnp.asarray(q.shape[-1], dtype=x.dtype))
    attn = jax.nn.softmax(q @ jnp.transpose(k, (0, 2, 1)) / scale, axis=-1)
    o = (attn @ v) @ a['wo'] + a['bo']
    return x + jnp.transpose(o, (0, 2, 1))


def _encode(x, p):
    z = x * p['w_proj'][None, :, None]
    for w, b in p['enc_conv']:
        z = jax.nn.relu(_conv1d(z, w, b))
    for a in p['enc_attn']:
        z = _attn_blk(z, a)
    return z


def _vq_body(zt_ref, emb_ref, e2_ref, idx_ref, zq_ref):
    f = zt_ref[0]
    f2 = jnp.sum(f * f, axis=1, keepdims=True)
    ids = lax.broadcasted_iota(jnp.int32, (_T, _KC), 1)

    def chunk(j, carry):
        bd, bi = carry
        e = emb_ref[pl.ds(j * _KC, _KC), :]
        g = _mm_t(f, e)
        e2 = e2_ref[0, pl.ds(j * _KC, _KC)][None, :]
        d = (f2 - 2.0 * g) + e2
        dmin = jnp.min(d, axis=1, keepdims=True)
        imin = jnp.min(jnp.where(d == dmin, ids, jnp.int32(2**31 - 1)),
                       axis=1, keepdims=True) + j * _KC
        better = dmin < bd
        return jnp.where(better, dmin, bd), jnp.where(better, imin, bi)

    bd0 = jnp.full((_T, 1), jnp.inf, jnp.float32)
    bi0 = jnp.zeros((_T, 1), jnp.int32)
    _, bi = lax.fori_loop(0, _K // _KC, chunk, (bd0, bi0))
    idx_ref[0] = bi

    def take(j, acc):
        e = emb_ref[pl.ds(j * _KC, _KC), :]
        m = jnp.where(ids == (bi - j * _KC), 1.0, 0.0)
        return acc + lax.dot_general(m, e, (((1,), (0,)), ((), ())),
                                     precision=lax.Precision.HIGHEST,
                                     preferred_element_type=jnp.float32)

    zq_ref[0] = lax.fori_loop(0, _K // _KC, take, jnp.zeros((_T, _D), jnp.float32))


def _gather_sc(table, idx):
    dp = 128
    tablep = jnp.pad(table, ((0, 0), (0, dp - table.shape[1])))
    info = plsc.get_sparse_core_info()
    nw = info.num_cores * info.num_subcores
    bt = idx.shape[0]
    bpw = bt // nw
    nc = info.num_cores
    mesh = plsc.VectorSubcoreMesh(core_axis_name="c", subcore_axis_name="s")

    @functools.partial(
        pl.kernel, mesh=mesh,
        out_type=jax.ShapeDtypeStruct((bt, dp), jnp.float32),
        scratch_types=[
            pltpu.VMEM((bpw,), jnp.int32),
            pltpu.VMEM((bpw, dp), jnp.float32),
            pltpu.SemaphoreType.DMA,
        ],
    )
    def k(table_hbm, idx_hbm, out_hbm, idx_v, rows_v, sem):
        wid = lax.axis_index("s") * nc + lax.axis_index("c")
        base = wid * bpw
        pltpu.sync_copy(idx_hbm.at[pl.ds(base, bpw)], idx_v)
        pltpu.async_copy(table_hbm.at[idx_v], rows_v, sem).wait()
        pltpu.sync_copy(rows_v, out_hbm.at[pl.ds(base, bpw)])

    return k(tablep, idx)[:, :_D]


def _conv3(h, w0, w1, w2, b):
    zrow = jnp.zeros((1, h.shape[1]), h.dtype)
    prev = jnp.concatenate([zrow, h[:-1]], axis=0)
    nxt = jnp.concatenate([h[1:], zrow], axis=0)
    y = _mm(prev, w0) + _mm(h, w1) + _mm(nxt, w2) + b
    return jnp.maximum(y, 0.0)


def _attn(h, wq, bq, wk, bk, wv, bv, wo, bo):
    q = _mm(h, wq) + bq
    k = _mm(h, wk) + bk
    v = _mm(h, wv) + bv
    s = _mm_t(q, k) * (1.0 / 8.0)
    m = jnp.max(s, axis=1, keepdims=True)
    e = jnp.exp(s - m)
    a = e / jnp.sum(e, axis=1, keepdims=True)
    o = _mm(_mm(a, v), wo) + bo
    return h + o


def _dec_body(zqt_ref,
              a0wq, a0bq, a0wk, a0bk, a0wv, a0bv, a0wo, a0bo,
              a1wq, a1bq, a1wk, a1bk, a1wv, a1bv, a1wo, a1bo,
              d1w0, d1w1, d1w2, d1b,
              d2w0, d2w1, d2w2, d2b,
              d3w0, d3w1, d3w2, d3b,
              rw_ref, rb_ref,
              out_ref):
    h = zqt_ref[0]
    h = _attn(h, a0wq[...], a0bq[...], a0wk[...], a0bk[...],
              a0wv[...], a0bv[...], a0wo[...], a0bo[...])
    h = _attn(h, a1wq[...], a1bq[...], a1wk[...], a1bk[...],
              a1wv[...], a1bv[...], a1wo[...], a1bo[...])
    h = _conv3(h, d1w0[...], d1w1[...], d1w2[...], d1b[...])
    h = _conv3(h, d2w0[...], d2w1[...], d2w2[...], d2b[...])
    h = _conv3(h, d3w0[...], d3w1[...], d3w2[...], d3b[...])
    out_ref[0] = _mm(h, rw_ref[...]) + rb_ref[...]


def _full_spec(arr):
    nd = arr.ndim
    return pl.BlockSpec(arr.shape, lambda b, _nd=nd: (0,) * _nd)


def _attn_flat(a):
    return [a['wq'], a['bq'].reshape(1, -1), a['wk'], a['bk'].reshape(1, -1),
            a['wv'], a['bv'].reshape(1, -1), a['wo'], a['bo'].reshape(1, -1)]


def _conv_flat(wb):
    w, b = wb
    return [w[:, :, 0].T, w[:, :, 1].T, w[:, :, 2].T, b.reshape(1, -1)]


def kernel(x, params):
    p = params
    emb = p['emb']

    z = _encode(x, p)
    zt = jnp.transpose(z, (0, 2, 1))
    e2 = jnp.sum(emb**2, axis=1)[None, :]

    idx3, zqt = pl.pallas_call(
        _vq_body,
        grid=(_B,),
        in_specs=[pl.BlockSpec((1, _T, _D), lambda b: (b, 0, 0)),
                  _full_spec(emb), _full_spec(e2)],
        out_specs=[pl.BlockSpec((1, _T, 1), lambda b: (b, 0, 0)),
                   pl.BlockSpec((1, _T, _D), lambda b: (b, 0, 0))],
        out_shape=[jax.ShapeDtypeStruct((_B, _T, 1), jnp.int32),
                   jax.ShapeDtypeStruct((_B, _T, _D), jnp.float32)],
    )(zt, emb, e2)

    indices = idx3.reshape(_B, _T)

    dec_in = [zqt]
    for a in p['dec_attn']:
        dec_in += _attn_flat(a)
    for wb in p['dec_conv']:
        dec_in += _conv_flat(wb)
    dec_in += [p['rw'], p['rb'].reshape(1, -1)]

    recont = pl.pallas_call(
        _dec_body,
        grid=(_B,),
        in_specs=[pl.BlockSpec((1, _T, _D), lambda b: (b, 0, 0))]
                 + [_full_spec(a) for a in dec_in[1:]],
        out_specs=pl.BlockSpec((1, _T, _NB), lambda b: (b, 0, 0)),
        out_shape=jax.ShapeDtypeStruct((_B, _T, _NB), jnp.float32),
    )(*dec_in)

    recon = jnp.transpose(recont, (0, 2, 1))
    zq = jnp.transpose(zqt, (0, 2, 1))
    return recon, z, zq, indices
